# Initial kernel scaffold; baseline (speedup 1.0000x reference)
import jax
import jax.numpy as jnp
from jax.experimental import pallas as pl


def kernel(log_probs, lengths):
    raise NotImplementedError("write your pallas kernel here")


# speedup vs baseline: 63.4303x; 63.4303x over previous
import functools
import jax
import jax.numpy as jnp
from jax import lax
from jax.experimental import pallas as pl
from jax.experimental.pallas import tpu as pltpu
from jax.experimental.pallas import tpu_sc as plsc

BLANK_ID = 0
NBEAM = 16
TBLK = 8


def _topk_body(x_ref, v_ref, c_ref):
    x = x_ref[...]
    tb, bb, cc = x.shape
    x = x.reshape(tb * bb, cc)
    iota = lax.broadcasted_iota(jnp.int32, (tb * bb, cc), 1)
    vs, cs = [], []
    for _ in range(NBEAM):
        m = jnp.max(x, axis=-1, keepdims=True)
        idx = jnp.min(jnp.where(x == m, iota, cc), axis=-1, keepdims=True)
        vs.append(m)
        cs.append(idx)
        x = jnp.where(iota == idx, -jnp.inf, x)
    v_ref[...] = jnp.concatenate(vs, axis=-1).reshape(tb, bb, NBEAM)
    c_ref[...] = jnp.concatenate(cs, axis=-1).reshape(tb, bb, NBEAM)


def _frame_topk(log_probs):
    T, B, C = log_probs.shape
    return pl.pallas_call(
        _topk_body,
        grid=(T // TBLK,),
        in_specs=[pl.BlockSpec((TBLK, B, C), lambda i: (i, 0, 0))],
        out_specs=[
            pl.BlockSpec((TBLK, B, NBEAM), lambda i: (i, 0, 0)),
            pl.BlockSpec((TBLK, B, NBEAM), lambda i: (i, 0, 0)),
        ],
        out_shape=[
            jax.ShapeDtypeStruct((T, B, NBEAM), jnp.float32),
            jax.ShapeDtypeStruct((T, B, NBEAM), jnp.int32),
        ],
    )(log_probs)


def _beam_body(tv_h, tc_h, len_h, score_h, dlen_h, dec_h,
               tv, tc, cls_s, bp_s, scv, path, dec_v, lenv, dlv, pid):
    T = dec_v.shape[0]
    b = lax.axis_index("s") * 2 + lax.axis_index("c")
    lane = lax.iota(jnp.int32, 16)
    zeros16 = jnp.zeros((16,), jnp.int32)

    pltpu.sync_copy(tv_h.at[b], tv)
    pltpu.sync_copy(tc_h.at[b], tc)
    pltpu.sync_copy(len_h, lenv)
    len_b = jnp.max(plsc.load_gather(lenv, [zeros16 + b]))

    sc0 = tv[pl.ds(0, 16)]
    cls_s[pl.ds(0, 16)] = tc[pl.ds(0, 16)]
    bp_s[pl.ds(0, 16)] = lane

    even_perm = lane ^ 1
    even_first = (lane & 1) == 0
    mid = (lane >= 1) & (lane <= 14)
    odd_perm = jnp.where(mid, lane + jnp.where((lane & 1) == 1, 1, -1), lane)
    odd_first = (lane & 1) == 1

    def tie_pass(R, P, perm, first):
        scv[...] = R
        pid[...] = P
        pR = plsc.load_gather(scv, [perm])
        pP = plsc.load_gather(pid, [perm])
        swap = (R == pR) & jnp.where(first, P > pP, P < pP)
        return jnp.where(swap, pR, R), jnp.where(swap, pP, P)

    def step(t, sc):
        scv[...] = sc
        v = tv[pl.ds(t * 16, 16)]
        c = tc[pl.ds(t * 16, 16)]
        v0 = jnp.max(v)
        R = plsc.load_gather(scv, [zeros16]) + v
        P = c

        def wcond(st):
            i, R, P = st
            sci = jnp.max(plsc.load_gather(scv, [zeros16 + jnp.minimum(i, 15)]))
            return (i < 16) & (sci + v0 > jnp.min(R))

        def wbody(st):
            i, R, P = st
            cand = jnp.max(plsc.load_gather(scv, [zeros16 + i])) + v
            rc = lax.rev(cand, (0,))
            rp = lax.rev(i * 1024 + c, (0,))
            take_r = R >= rc
            hi = jnp.maximum(R, rc)
            hp = jnp.where(take_r, P, rp)
            hi, hp = plsc.sort_key_val(hi, hp, descending=True)
            return (i + 1, hi, hp)

        _, R, P = lax.while_loop(wcond, wbody, (jnp.int32(1), R, P))
        R, P = tie_pass(R, P, even_perm, even_first)
        R, P = tie_pass(R, P, odd_perm, odd_first)
        R, P = tie_pass(R, P, even_perm, even_first)
        bp_s[pl.ds(t * 16, 16)] = lax.shift_right_logical(P, 10)
        cls_s[pl.ds(t * 16, 16)] = P & 1023
        return R

    sc_fin = lax.fori_loop(1, len_b, step, sc0)

    lane0 = lane == 0

    def bt(k, bm):
        t = len_b - 1 - k
        clsv = plsc.load_gather(cls_s, [t * 16 + bm])
        plsc.store_scatter(path, [zeros16 + t], clsv, mask=lane0)
        return plsc.load_gather(bp_s, [t * 16 + bm])

    lax.fori_loop(0, len_b, bt, zeros16)

    for ci in range(T // 16):
        dec_v[pl.ds(ci * 16, 16)] = zeros16

    def collapse(ci, cnt):
        tvec = ci * 16 + lane
        pc = path[pl.ds(ci * 16, 16)]
        prv = plsc.load_gather(path, [jnp.maximum(tvec - 1, 0)])
        prv = jnp.where(tvec == 0, -1, prv)
        keep = (pc != BLANK_ID) & (pc != prv) & (tvec < len_b)
        pos = plsc.cumsum(jnp.where(keep, 1, 0)) + cnt
        plsc.store_scatter(dec_v, [jnp.maximum(pos - 1, 0)], pc, mask=keep)
        return jnp.max(pos)

    cnt = lax.fori_loop(0, T // 16, collapse, jnp.int32(0))

    scv[...] = sc_fin
    dlv[...] = zeros16 + cnt
    pltpu.sync_copy(scv, score_h.at[b])
    pltpu.sync_copy(dlv, dlen_h.at[b])
    pltpu.sync_copy(dec_v, dec_h.at[b])


def _beam_search(topv, topc, lengths):
    B, T, _ = topv.shape
    mesh = plsc.VectorSubcoreMesh(core_axis_name="c", subcore_axis_name="s")
    run = functools.partial(
        pl.kernel,
        mesh=mesh,
        compiler_params=pltpu.CompilerParams(needs_layout_passes=False),
        out_type=[
            jax.ShapeDtypeStruct((B, 16), jnp.float32),
            jax.ShapeDtypeStruct((B, 16), jnp.int32),
            jax.ShapeDtypeStruct((B, T), jnp.int32),
        ],
        scratch_types=[
            pltpu.VMEM((T * 16,), jnp.float32),
            pltpu.VMEM((T * 16,), jnp.int32),
            pltpu.VMEM((T * 16,), jnp.int32),
            pltpu.VMEM((T * 16,), jnp.int32),
            pltpu.VMEM((16,), jnp.float32),
            pltpu.VMEM((T,), jnp.int32),
            pltpu.VMEM((T,), jnp.int32),
            pltpu.VMEM((32,), jnp.int32),
            pltpu.VMEM((16,), jnp.int32),
            pltpu.VMEM((16,), jnp.int32),
        ],
    )(_beam_body)
    return run(topv.reshape(B, T * 16), topc.reshape(B, T * 16), lengths)


def kernel(log_probs, lengths):
    T, B, C = log_probs.shape
    topv, topc = _frame_topk(log_probs)
    topv = jnp.transpose(topv, (1, 0, 2))
    topc = jnp.transpose(topc, (1, 0, 2))
    score, dlen, dec = _beam_search(topv, topc, lengths)
    return score[:, :1], dlen[:, :1], dec[:, None, :]


# speedup vs baseline: 114.9554x; 1.8123x over previous
import functools
import jax
import jax.numpy as jnp
from jax import lax
from jax.experimental import pallas as pl
from jax.experimental.pallas import tpu as pltpu
from jax.experimental.pallas import tpu_sc as plsc

BLANK_ID = 0
NBEAM = 16
TBLK = 16


def _topk_body(x_ref, v_ref, c_ref):
    x = x_ref[...]
    tb, bb, cc = x.shape
    x = x.reshape(tb * bb, cc)
    iota = lax.broadcasted_iota(jnp.int32, (tb * bb, cc), 1)
    lane16 = lax.broadcasted_iota(jnp.int32, (tb * bb, NBEAM), 1)
    vacc = jnp.zeros((tb * bb, NBEAM), jnp.float32)
    cacc = jnp.zeros((tb * bb, NBEAM), jnp.int32)
    for k in range(NBEAM):
        m = jnp.max(x, axis=-1, keepdims=True)
        idx = jnp.min(jnp.where(x == m, iota, cc), axis=-1, keepdims=True)
        vacc = jnp.where(lane16 == k, m, vacc)
        cacc = jnp.where(lane16 == k, idx, cacc)
        x = jnp.where(iota == idx, -jnp.inf, x)
    v_ref[...] = vacc.reshape(tb, bb, NBEAM)
    c_ref[...] = cacc.reshape(tb, bb, NBEAM)


def _frame_topk(log_probs):
    T, B, C = log_probs.shape
    return pl.pallas_call(
        _topk_body,
        grid=(T // TBLK,),
        in_specs=[pl.BlockSpec((TBLK, B, C), lambda i: (i, 0, 0))],
        out_specs=[
            pl.BlockSpec((TBLK, B, NBEAM), lambda i: (i, 0, 0)),
            pl.BlockSpec((TBLK, B, NBEAM), lambda i: (i, 0, 0)),
        ],
        out_shape=[
            jax.ShapeDtypeStruct((T, B, NBEAM), jnp.float32),
            jax.ShapeDtypeStruct((T, B, NBEAM), jnp.int32),
        ],
    )(log_probs)


def _beam_body(tv_h, tc_h, len_h, score_h, dlen_h, dec_h,
               tv, tc, cls_s, bp_s, scv, path, dec_v, lenv, dlv, pid):
    T = dec_v.shape[0]
    b = lax.axis_index("s") * 2 + lax.axis_index("c")
    lane = lax.iota(jnp.int32, 16)
    zeros16 = jnp.zeros((16,), jnp.int32)

    pltpu.sync_copy(tv_h.at[b], tv)
    pltpu.sync_copy(tc_h.at[b], tc)
    pltpu.sync_copy(len_h, lenv)
    len_b = jnp.max(plsc.load_gather(lenv, [zeros16 + b]))

    sc0 = tv[pl.ds(0, 16)]
    cls_s[pl.ds(0, 16)] = tc[pl.ds(0, 16)]
    bp_s[pl.ds(0, 16)] = lane

    even_perm = lane ^ 1
    even_first = (lane & 1) == 0
    mid = (lane >= 1) & (lane <= 14)
    odd_perm = jnp.where(mid, lane + jnp.where((lane & 1) == 1, 1, -1), lane)
    odd_first = (lane & 1) == 1

    def tie_pass(R, P, perm, first):
        scv[...] = R
        pid[...] = P
        pR = plsc.load_gather(scv, [perm])
        pP = plsc.load_gather(pid, [perm])
        swap = (R == pR) & jnp.where(first, P > pP, P < pP)
        return jnp.where(swap, pR, R), jnp.where(swap, pP, P)

    def step(t, sc):
        scv[...] = sc
        v = tv[pl.ds(t * 16, 16)]
        c = tc[pl.ds(t * 16, 16)]
        sc0s = plsc.load_gather(scv, [zeros16])
        R = sc0s + v
        P = c
        rvc = lax.rev(c, (0,))
        rv = lax.rev(v, (0,))

        for i in range(1, 16):
            rc = plsc.load_gather(scv, [zeros16 + i]) + rv
            rp = i * 1024 + rvc
            take_r = R >= rc
            hi = jnp.maximum(R, rc)
            hp = jnp.where(take_r, P, rp)
            R, P = plsc.sort_key_val(hi, hp, descending=True)
        R, P = tie_pass(R, P, even_perm, even_first)
        R, P = tie_pass(R, P, odd_perm, odd_first)
        R, P = tie_pass(R, P, even_perm, even_first)
        bp_s[pl.ds(t * 16, 16)] = lax.shift_right_logical(P, 10)
        cls_s[pl.ds(t * 16, 16)] = P & 1023
        return R

    sc_fin = lax.fori_loop(1, len_b, step, sc0)

    lane0 = lane == 0

    def bt(k, bm):
        t = len_b - 1 - k
        clsv = plsc.load_gather(cls_s, [t * 16 + bm])
        plsc.store_scatter(path, [zeros16 + t], clsv, mask=lane0)
        return plsc.load_gather(bp_s, [t * 16 + bm])

    lax.fori_loop(0, len_b, bt, zeros16)

    for ci in range(T // 16):
        dec_v[pl.ds(ci * 16, 16)] = zeros16

    def collapse(ci, cnt):
        tvec = ci * 16 + lane
        pc = path[pl.ds(ci * 16, 16)]
        prv = plsc.load_gather(path, [jnp.maximum(tvec - 1, 0)])
        prv = jnp.where(tvec == 0, -1, prv)
        keep = (pc != BLANK_ID) & (pc != prv) & (tvec < len_b)
        pos = plsc.cumsum(jnp.where(keep, 1, 0)) + cnt
        plsc.store_scatter(dec_v, [jnp.maximum(pos - 1, 0)], pc, mask=keep)
        return jnp.max(pos)

    cnt = lax.fori_loop(0, T // 16, collapse, jnp.int32(0))

    scv[...] = sc_fin
    dlv[...] = zeros16 + cnt
    pltpu.sync_copy(scv, score_h.at[b])
    pltpu.sync_copy(dlv, dlen_h.at[b])
    pltpu.sync_copy(dec_v, dec_h.at[b])


def _beam_search(topv, topc, lengths):
    B, T, _ = topv.shape
    mesh = plsc.VectorSubcoreMesh(core_axis_name="c", subcore_axis_name="s")
    run = functools.partial(
        pl.kernel,
        mesh=mesh,
        compiler_params=pltpu.CompilerParams(needs_layout_passes=False),
        out_type=[
            jax.ShapeDtypeStruct((B, 16), jnp.float32),
            jax.ShapeDtypeStruct((B, 16), jnp.int32),
            jax.ShapeDtypeStruct((B, T), jnp.int32),
        ],
        scratch_types=[
            pltpu.VMEM((T * 16,), jnp.float32),
            pltpu.VMEM((T * 16,), jnp.int32),
            pltpu.VMEM((T * 16,), jnp.int32),
            pltpu.VMEM((T * 16,), jnp.int32),
            pltpu.VMEM((16,), jnp.float32),
            pltpu.VMEM((T,), jnp.int32),
            pltpu.VMEM((T,), jnp.int32),
            pltpu.VMEM((32,), jnp.int32),
            pltpu.VMEM((16,), jnp.int32),
            pltpu.VMEM((16,), jnp.int32),
        ],
    )(_beam_body)
    return run(topv.reshape(B, T * 16), topc.reshape(B, T * 16), lengths)


def kernel(log_probs, lengths):
    T, B, C = log_probs.shape
    topv, topc = _frame_topk(log_probs)
    topv = jnp.transpose(topv, (1, 0, 2))
    topc = jnp.transpose(topc, (1, 0, 2))
    score, dlen, dec = _beam_search(topv, topc, lengths)
    return score[:, :1], dlen[:, :1], dec[:, None, :]


# speedup vs baseline: 118.8326x; 1.0337x over previous
import functools
import jax
import jax.numpy as jnp
from jax import lax
from jax.experimental import pallas as pl
from jax.experimental.pallas import tpu as pltpu
from jax.experimental.pallas import tpu_sc as plsc

BLANK_ID = 0
NBEAM = 16
TBLK = 16


def _topk_body(x_ref, c_ref):
    x = x_ref[...]
    tb, bb, cc = x.shape
    x = x.reshape(tb * bb, cc)
    iota = lax.broadcasted_iota(jnp.int32, (tb * bb, cc), 1)
    lane16 = lax.broadcasted_iota(jnp.int32, (tb * bb, NBEAM), 1)
    cacc = jnp.zeros((tb * bb, NBEAM), jnp.int32)
    for k in range(NBEAM):
        idx = jnp.argmax(x, axis=-1).astype(jnp.int32)[:, None]
        cacc = jnp.where(lane16 == k, idx, cacc)
        x = jnp.where(iota == idx, -jnp.inf, x)
    c_ref[...] = cacc.reshape(tb, bb, NBEAM)


def _frame_topk(log_probs):
    T, B, C = log_probs.shape
    return pl.pallas_call(
        _topk_body,
        grid=(T // TBLK,),
        in_specs=[pl.BlockSpec((TBLK, B, C), lambda i: (i, 0, 0))],
        out_specs=pl.BlockSpec((TBLK, B, NBEAM), lambda i: (i, 0, 0)),
        out_shape=jax.ShapeDtypeStruct((T, B, NBEAM), jnp.int32),
    )(log_probs)


def _beam_body(tv_h, tc_h, len_h, score_h, dlen_h, dec_h,
               tv, tc, cls_s, bp_s, scv, path, dec_v, lenv, dlv, pid):
    T = dec_v.shape[0]
    b = lax.axis_index("s") * 2 + lax.axis_index("c")
    lane = lax.iota(jnp.int32, 16)
    zeros16 = jnp.zeros((16,), jnp.int32)

    pltpu.sync_copy(tv_h.at[b], tv)
    pltpu.sync_copy(tc_h.at[b], tc)
    pltpu.sync_copy(len_h, lenv)
    len_b = jnp.max(plsc.load_gather(lenv, [zeros16 + b]))

    sc0 = tv[pl.ds(0, 16)]
    cls_s[pl.ds(0, 16)] = tc[pl.ds(0, 16)]
    bp_s[pl.ds(0, 16)] = lane

    even_perm = lane ^ 1
    even_first = (lane & 1) == 0
    mid = (lane >= 1) & (lane <= 14)
    odd_perm = jnp.where(mid, lane + jnp.where((lane & 1) == 1, 1, -1), lane)
    odd_first = (lane & 1) == 1

    def tie_pass(R, P, perm, first):
        scv[...] = R
        pid[...] = P
        pR = plsc.load_gather(scv, [perm])
        pP = plsc.load_gather(pid, [perm])
        swap = (R == pR) & jnp.where(first, P > pP, P < pP)
        return jnp.where(swap, pR, R), jnp.where(swap, pP, P)

    def step(t, sc):
        scv[...] = sc
        v = tv[pl.ds(t * 16, 16)]
        c = tc[pl.ds(t * 16, 16)]
        sc0s = plsc.load_gather(scv, [zeros16])
        R = sc0s + v
        P = c
        rvc = lax.rev(c, (0,))
        rv = lax.rev(v, (0,))

        for i in range(1, 16):
            rc = plsc.load_gather(scv, [zeros16 + i]) + rv
            rp = i * 1024 + rvc
            take_r = R >= rc
            hi = jnp.maximum(R, rc)
            hp = jnp.where(take_r, P, rp)
            R, P = plsc.sort_key_val(hi, hp, descending=True)
        R, P = tie_pass(R, P, even_perm, even_first)
        R, P = tie_pass(R, P, odd_perm, odd_first)
        R, P = tie_pass(R, P, even_perm, even_first)
        bp_s[pl.ds(t * 16, 16)] = lax.shift_right_logical(P, 10)
        cls_s[pl.ds(t * 16, 16)] = P & 1023
        return R

    sc_fin = lax.fori_loop(1, len_b, step, sc0)

    lane0 = lane == 0

    def bt(k, bm):
        t = len_b - 1 - k
        clsv = plsc.load_gather(cls_s, [t * 16 + bm])
        plsc.store_scatter(path, [zeros16 + t], clsv, mask=lane0)
        return plsc.load_gather(bp_s, [t * 16 + bm])

    lax.fori_loop(0, len_b, bt, zeros16)

    for ci in range(T // 16):
        dec_v[pl.ds(ci * 16, 16)] = zeros16

    def collapse(ci, cnt):
        tvec = ci * 16 + lane
        pc = path[pl.ds(ci * 16, 16)]
        prv = plsc.load_gather(path, [jnp.maximum(tvec - 1, 0)])
        prv = jnp.where(tvec == 0, -1, prv)
        keep = (pc != BLANK_ID) & (pc != prv) & (tvec < len_b)
        pos = plsc.cumsum(jnp.where(keep, 1, 0)) + cnt
        plsc.store_scatter(dec_v, [jnp.maximum(pos - 1, 0)], pc, mask=keep)
        return jnp.max(pos)

    cnt = lax.fori_loop(0, T // 16, collapse, jnp.int32(0))

    scv[...] = sc_fin
    dlv[...] = zeros16 + cnt
    pltpu.sync_copy(scv, score_h.at[b])
    pltpu.sync_copy(dlv, dlen_h.at[b])
    pltpu.sync_copy(dec_v, dec_h.at[b])


def _beam_search(topv, topc, lengths):
    B, T, _ = topv.shape
    mesh = plsc.VectorSubcoreMesh(core_axis_name="c", subcore_axis_name="s")
    run = functools.partial(
        pl.kernel,
        mesh=mesh,
        compiler_params=pltpu.CompilerParams(needs_layout_passes=False),
        out_type=[
            jax.ShapeDtypeStruct((B, 16), jnp.float32),
            jax.ShapeDtypeStruct((B, 16), jnp.int32),
            jax.ShapeDtypeStruct((B, T), jnp.int32),
        ],
        scratch_types=[
            pltpu.VMEM((T * 16,), jnp.float32),
            pltpu.VMEM((T * 16,), jnp.int32),
            pltpu.VMEM((T * 16,), jnp.int32),
            pltpu.VMEM((T * 16,), jnp.int32),
            pltpu.VMEM((16,), jnp.float32),
            pltpu.VMEM((T,), jnp.int32),
            pltpu.VMEM((T,), jnp.int32),
            pltpu.VMEM((32,), jnp.int32),
            pltpu.VMEM((16,), jnp.int32),
            pltpu.VMEM((16,), jnp.int32),
        ],
    )(_beam_body)
    return run(topv.reshape(B, T * 16), topc.reshape(B, T * 16), lengths)


def kernel(log_probs, lengths):
    T, B, C = log_probs.shape
    topc = _frame_topk(log_probs)
    topv = jnp.take_along_axis(log_probs, topc, axis=-1)
    topv = jnp.transpose(topv, (1, 0, 2))
    topc = jnp.transpose(topc, (1, 0, 2))
    score, dlen, dec = _beam_search(topv, topc, lengths)
    return score[:, :1], dlen[:, :1], dec[:, None, :]
